# baseline (speedup 1.0000x reference)
import functools

import jax
import jax.numpy as jnp
from jax import lax
from jax.experimental import pallas as pl
from jax.experimental.pallas import tpu as pltpu
from jax.experimental.pallas import tpu_sc as plsc

B = 16384
D = 64
L = 16
NC = 2
NS = 16
NW = NC * NS
BPW = B // NW
NQ = 4
QT = BPW // NQ
QG = QT // L


def _sc_body(hidx_hbm, ridx_hbm, tidx_hbm, ent_hbm, rel_hbm, out_hbm,
             hidx, ridx, tidx,
             hrows0, rrows0, trows0, hrows1, rrows1, trows1,
             out_v, sem0, sem1):
    wid = lax.axis_index("s") * NC + lax.axis_index("c")
    base = wid * BPW

    sl = pl.ds(base, BPW)
    pltpu.sync_copy(hidx_hbm.at[sl], hidx)
    pltpu.sync_copy(ridx_hbm.at[sl], ridx)
    pltpu.sync_copy(tidx_hbm.at[sl], tidx)

    bufs = ((hrows0, rrows0, trows0, sem0), (hrows1, rrows1, trows1, sem1))
    lane = lax.iota(jnp.int32, L)

    def transfers(q, hr, rr, tr, sem):
        q0 = q * QT

        def run(grp, start):
            t0 = grp * L
            iv_h = hidx[pl.ds(q0 + t0, L)]
            iv_r = ridx[pl.ds(q0 + t0, L)]
            iv_t = tidx[pl.ds(q0 + t0, L)]
            for k in range(L):
                t = t0 + k
                ch = pltpu.make_async_copy(ent_hbm.at[iv_h[k]], hr.at[t], sem)
                cr = pltpu.make_async_copy(rel_hbm.at[iv_r[k]], rr.at[t], sem)
                ct = pltpu.make_async_copy(ent_hbm.at[iv_t[k]], tr.at[t], sem)
                for c in (ch, cr, ct):
                    if start:
                        c.start()
                    else:
                        c.wait()
        return run

    def issue(q):
        hr, rr, tr, sem = bufs[q % 2]
        run = transfers(q, hr, rr, tr, sem)
        lax.fori_loop(0, QG, lambda g, c: (run(g, True), c)[1], 0)

    def drain(q):
        hr, rr, tr, sem = bufs[q % 2]
        run = transfers(q, hr, rr, tr, sem)
        lax.fori_loop(0, QG, lambda g, c: (run(g, False), c)[1], 0)

    def compute(q):
        hr, rr, tr, _ = bufs[q % 2]

        def body(grp, carry):
            tvec = lane + grp * L
            acc = jnp.zeros((L,), jnp.float32)
            for d in range(D):
                dvec = jnp.full((L,), d, jnp.int32)
                hv = plsc.load_gather(hr, [tvec, dvec])
                rv = plsc.load_gather(rr, [tvec, dvec])
                tv = plsc.load_gather(tr, [tvec, dvec])
                acc = acc + hv * rv * tv
            out_v[pl.ds(q * QT + grp * L, L)] = acc
            return carry
        lax.fori_loop(0, QG, body, 0)

    issue(0)
    for q in range(NQ):
        if q + 1 < NQ:
            issue(q + 1)
        drain(q)
        compute(q)

    pltpu.sync_copy(out_v, out_hbm.at[pl.ds(base, BPW)])


@jax.jit
def _sc_score(head_indices, rel_indices, tail_indices, ent, rel):
    run = functools.partial(
        pl.kernel,
        mesh=plsc.VectorSubcoreMesh(core_axis_name="c", subcore_axis_name="s"),
        compiler_params=pltpu.CompilerParams(
            needs_layout_passes=False, use_tc_tiling_on_sc=True),
        out_type=jax.ShapeDtypeStruct((B,), jnp.float32),
        scratch_types=[
            pltpu.VMEM((BPW,), jnp.int32),
            pltpu.VMEM((BPW,), jnp.int32),
            pltpu.VMEM((BPW,), jnp.int32),
            pltpu.VMEM((QT, D), jnp.float32),
            pltpu.VMEM((QT, D), jnp.float32),
            pltpu.VMEM((QT, D), jnp.float32),
            pltpu.VMEM((QT, D), jnp.float32),
            pltpu.VMEM((QT, D), jnp.float32),
            pltpu.VMEM((QT, D), jnp.float32),
            pltpu.VMEM((BPW,), jnp.float32),
            pltpu.SemaphoreType.DMA,
            pltpu.SemaphoreType.DMA,
        ],
    )(_sc_body)
    return run(head_indices, rel_indices, tail_indices, ent, rel)


def kernel(head_indices, rel_indices, tail_indices, entity_embedding, relation_embedding):
    scores = _sc_score(head_indices, rel_indices, tail_indices,
                       entity_embedding, relation_embedding)
    return scores.reshape(B, 1)
